# baseline (speedup 1.0000x reference)
import jax
import jax.numpy as jnp
from jax import lax
from jax.experimental import pallas as pl
from jax.experimental.pallas import tpu as pltpu
from jax.experimental.pallas import tpu_sc as plsc

_D = 768
_NC = 2
_NS = 16
_NW = _NC * _NS
_CHUNK = 50
_CA = 48
_CB = 8
_NCHUNK = 128
_NBUF = 2


def _emb_body(idxa_hbm, idxb_hbm, table_hbm, out_hbm,
              idxa_v, idxb_v, rows0, rows1, tail0, tail1,
              gsem0, gsem1, tsem0, tsem1, osem0, osem1):
    wid = lax.axis_index("s") * _NC + lax.axis_index("c")
    base = wid * _NCHUNK
    pltpu.sync_copy(idxa_hbm.at[wid], idxa_v)
    pltpu.sync_copy(idxb_hbm.at[wid], idxb_v)

    rows = (rows0, rows1)
    tail = (tail0, tail1)
    gsem = (gsem0, gsem1)
    tsem = (tsem0, tsem1)
    osem = (osem0, osem1)

    def start_gather(g, b):
        pltpu.make_async_copy(
            table_hbm.at[idxa_v.at[g]], rows[b].at[pl.ds(0, _CA)],
            gsem[b]).start()
        pltpu.make_async_copy(
            table_hbm.at[idxb_v.at[g]], tail[b], tsem[b]).start()

    def wait_gather(b):
        pltpu.make_async_copy(
            table_hbm.at[idxa_v.at[0]], rows[b].at[pl.ds(0, _CA)],
            gsem[b]).wait()
        pltpu.make_async_copy(
            table_hbm.at[idxb_v.at[0]], tail[b], tsem[b]).wait()

    def start_out(g, b):
        for r in range(_CHUNK - _CA):
            for j in range(_D // 16):
                rows[b][_CA + r, pl.ds(16 * j, 16)] = (
                    tail[b][r, pl.ds(16 * j, 16)])
        pltpu.make_async_copy(rows[b], out_hbm.at[base + g], osem[b]).start()

    def wait_out(b):
        pltpu.make_async_copy(rows[b], out_hbm.at[base], osem[b]).wait()

    for b in range(_NBUF):
        start_gather(b, b)

    def ring(i, carry):
        for b in range(_NBUF):
            g = i * _NBUF + b
            bprev = (b - 1) % _NBUF
            gprev = g - 1

            @pl.when(gprev >= 0)
            def _():
                wait_out(bprev)

            @pl.when((gprev >= 0) & (gprev + _NBUF < _NCHUNK))
            def _():
                start_gather(gprev + _NBUF, bprev)

            wait_gather(b)
            start_out(g, b)
        return carry

    lax.fori_loop(0, _NCHUNK // _NBUF, ring, 0, unroll=False)
    wait_out(_NBUF - 1)


@jax.jit
def _emb_call(idxa, idxb, weight):
    mesh = plsc.VectorSubcoreMesh(core_axis_name="c", subcore_axis_name="s")
    return pl.kernel(
        _emb_body,
        out_type=jax.ShapeDtypeStruct((_NW * _NCHUNK, _CHUNK, _D),
                                      jnp.float32),
        mesh=mesh,
        scratch_types=(
            [pltpu.VMEM((_NCHUNK, _CA), jnp.int32),
             pltpu.VMEM((_NCHUNK, _CB), jnp.int32)]
            + [pltpu.VMEM((_CHUNK, _D), jnp.float32)] * _NBUF
            + [pltpu.VMEM((_CB, _D), jnp.float32)] * _NBUF
            + [pltpu.SemaphoreType.DMA] * (3 * _NBUF)
        ),
    )(idxa, idxb, weight)


def kernel(input_ids, weight):
    idx = input_ids.reshape(_NW, _NCHUNK, _CHUNK).astype(jnp.int32)
    idxa = idx[:, :, :_CA]
    idxb = jnp.pad(idx[:, :, _CA:], ((0, 0), (0, 0), (0, _CB - (_CHUNK - _CA))))
    return _emb_call(idxa, idxb, weight)


# speedup vs baseline: 1.0267x; 1.0267x over previous
import jax
import jax.numpy as jnp
from jax import lax
from jax.experimental import pallas as pl
from jax.experimental.pallas import tpu as pltpu
from jax.experimental.pallas import tpu_sc as plsc

_D = 768
_NC = 2
_NS = 16
_NW = _NC * _NS
_CHUNK = 50
_CA = 48
_CB = 8
_NT = 2
_NCHUNK = 128
_NBUF = 2


def _emb_body(idxa_hbm, idxb_hbm, table_hbm, out_hbm,
              idxa_v, idxb_v, rows0, rows1, tail0, tail1,
              gsem0, gsem1, tsem0, tsem1, osem0, osem1, usem0, usem1):
    wid = lax.axis_index("s") * _NC + lax.axis_index("c")
    base = wid * _NCHUNK
    pltpu.sync_copy(idxa_hbm.at[wid], idxa_v)
    pltpu.sync_copy(idxb_hbm.at[wid], idxb_v)

    rows = (rows0, rows1)
    tail = (tail0, tail1)
    gsem = (gsem0, gsem1)
    tsem = (tsem0, tsem1)
    osem = (osem0, osem1)
    usem = (usem0, usem1)

    def start_gather(g, b):
        pltpu.make_async_copy(
            table_hbm.at[idxa_v.at[g]], rows[b].at[pl.ds(0, _CA)],
            gsem[b]).start()
        pltpu.make_async_copy(
            table_hbm.at[idxb_v.at[g]], tail[b], tsem[b]).start()

    def wait_gather(b):
        pltpu.make_async_copy(
            table_hbm.at[idxa_v.at[0]], rows[b].at[pl.ds(0, _CA)],
            gsem[b]).wait()
        pltpu.make_async_copy(
            table_hbm.at[idxb_v.at[0]], tail[b], tsem[b]).wait()

    def start_out(g, b):
        pltpu.make_async_copy(rows[b], out_hbm.at[base + g, pl.ds(0, _CA)],
                              osem[b]).start()
        for r in range(_NT):
            pltpu.make_async_copy(tail[b].at[r],
                                  out_hbm.at[base + g, _CA + r],
                                  usem[b]).start()

    def wait_out(b):
        pltpu.make_async_copy(rows[b], out_hbm.at[base, pl.ds(0, _CA)],
                              osem[b]).wait()
        for r in range(_NT):
            pltpu.make_async_copy(tail[b].at[r], out_hbm.at[base, _CA + r],
                                  usem[b]).wait()

    for b in range(_NBUF):
        start_gather(b, b)

    def ring(i, carry):
        for b in range(_NBUF):
            g = i * _NBUF + b
            bprev = (b - 1) % _NBUF
            gprev = g - 1

            @pl.when(gprev >= 0)
            def _():
                wait_out(bprev)

            @pl.when((gprev >= 0) & (gprev + _NBUF < _NCHUNK))
            def _():
                start_gather(gprev + _NBUF, bprev)

            wait_gather(b)
            start_out(g, b)
        return carry

    lax.fori_loop(0, _NCHUNK // _NBUF, ring, 0, unroll=False)
    wait_out(_NBUF - 1)


@jax.jit
def _emb_call(idxa, idxb, weight):
    mesh = plsc.VectorSubcoreMesh(core_axis_name="c", subcore_axis_name="s")
    return pl.kernel(
        _emb_body,
        out_type=jax.ShapeDtypeStruct((_NW * _NCHUNK, _CHUNK, _D),
                                      jnp.float32),
        mesh=mesh,
        scratch_types=(
            [pltpu.VMEM((_NCHUNK, _CA), jnp.int32),
             pltpu.VMEM((_NCHUNK, _CB), jnp.int32)]
            + [pltpu.VMEM((_CA, _D), jnp.float32)] * _NBUF
            + [pltpu.VMEM((_CB, _D), jnp.float32)] * _NBUF
            + [pltpu.SemaphoreType.DMA] * (4 * _NBUF)
        ),
    )(idxa, idxb, weight)


def kernel(input_ids, weight):
    idx = input_ids.reshape(_NW, _NCHUNK, _CHUNK).astype(jnp.int32)
    idxa = idx[:, :, :_CA]
    idxb = jnp.pad(idx[:, :, _CA:], ((0, 0), (0, 0), (0, _CB - _NT)))
    return _emb_call(idxa, idxb, weight)


# speedup vs baseline: 2.8857x; 2.8107x over previous
import jax
import jax.numpy as jnp
from jax import lax
from jax.experimental import pallas as pl
from jax.experimental.pallas import tpu as pltpu
from jax.experimental.pallas import tpu_sc as plsc

_D = 768
_NC = 2
_NS = 16
_NW = _NC * _NS
_CHUNK = 50
_CA = 48
_NT = 2
_NCHUNK = 128
_NBUF = 2
_SBS = 8
_NSB = _NCHUNK // _SBS
_TROWS = _SBS * _NT


def _emb_body(idxa_hbm, idxb_hbm, table_hbm, out_hbm,
              idxa_v, idxb_v, rows0, rows1, tb0, tb1,
              gsem0, gsem1, osem0, osem1,
              tgsem0, tgsem1, usem0, usem1):
    wid = lax.axis_index("s") * _NC + lax.axis_index("c")
    base = wid * _NCHUNK
    pltpu.sync_copy(idxa_hbm.at[wid], idxa_v)
    pltpu.sync_copy(idxb_hbm.at[wid], idxb_v)

    rows = (rows0, rows1)
    tbuf = (tb0, tb1)
    gsem = (gsem0, gsem1)
    osem = (osem0, osem1)
    tgsem = (tgsem0, tgsem1)
    usem = (usem0, usem1)

    def start_head(g, b):
        pltpu.make_async_copy(
            table_hbm.at[idxa_v.at[g]], rows[b], gsem[b]).start()

    def wait_head(b):
        pltpu.make_async_copy(
            table_hbm.at[idxa_v.at[0]], rows[b], gsem[b]).wait()

    def start_out(g, b):
        pltpu.make_async_copy(rows[b], out_hbm.at[base + g, pl.ds(0, _CA)],
                              osem[b]).start()

    def wait_out(b):
        pltpu.make_async_copy(rows[b], out_hbm.at[base, pl.ds(0, _CA)],
                              osem[b]).wait()

    def start_tails(sb, t):
        pltpu.make_async_copy(
            table_hbm.at[idxb_v.at[sb]], tbuf[t], tgsem[t]).start()

    def wait_tails(t):
        pltpu.make_async_copy(
            table_hbm.at[idxb_v.at[0]], tbuf[t], tgsem[t]).wait()

    def drain_tail_writes(t):
        for _ in range(_TROWS):
            pltpu.make_async_copy(tbuf[t].at[0], out_hbm.at[base, _CA],
                                  usem[t]).wait()

    start_head(0, 0)
    start_head(1, 1)
    start_tails(0, 0)
    start_tails(1, 1)

    def outer(i, carry):
        for t in range(2):
            sb = 2 * i + t
            wait_tails(t)
            for k in range(_SBS):
                g = sb * _SBS + k
                b = k % _NBUF
                bprev = (b - 1) % _NBUF
                gprev = g - 1

                @pl.when(gprev >= 0)
                def _():
                    wait_out(bprev)

                @pl.when((gprev >= 0) & (gprev + _NBUF < _NCHUNK))
                def _():
                    start_head(gprev + _NBUF, bprev)

                wait_head(b)
                start_out(g, b)
                for r in range(_NT):
                    pltpu.make_async_copy(tbuf[t].at[_NT * k + r],
                                          out_hbm.at[base + g, _CA + r],
                                          usem[t]).start()
            drain_tail_writes(t)

            @pl.when(sb + 2 < _NSB)
            def _():
                start_tails(sb + 2, t)
        return carry

    lax.fori_loop(0, _NSB // 2, outer, 0, unroll=False)
    wait_out(1)


@jax.jit
def _emb_call(idxa, idxb, weight):
    mesh = plsc.VectorSubcoreMesh(core_axis_name="c", subcore_axis_name="s")
    return pl.kernel(
        _emb_body,
        out_type=jax.ShapeDtypeStruct((_NW * _NCHUNK, _CHUNK, _D),
                                      jnp.float32),
        mesh=mesh,
        scratch_types=(
            [pltpu.VMEM((_NCHUNK, _CA), jnp.int32),
             pltpu.VMEM((_NSB, _TROWS), jnp.int32)]
            + [pltpu.VMEM((_CA, _D), jnp.float32)] * _NBUF
            + [pltpu.VMEM((_TROWS, _D), jnp.float32)] * 2
            + [pltpu.SemaphoreType.DMA] * 8
        ),
    )(idxa, idxb, weight)


def kernel(input_ids, weight):
    idx = input_ids.reshape(_NW, _NCHUNK, _CHUNK).astype(jnp.int32)
    idxa = idx[:, :, :_CA]
    idxb = idx[:, :, _CA:].reshape(_NW, _NSB, _TROWS)
    return _emb_call(idxa, idxb, weight)
